# baseline (speedup 1.0000x reference)
import jax
import jax.numpy as jnp
from jax.experimental import pallas as pl
from jax.experimental.pallas import tpu as pltpu

L = 2048; HID = 2048; H = 4; D = 128; DV = 128; N = 4; LR = 16
GLN = 16.0; EPS = 1e-5
HD = H * D
C = 256
TL = 256
CPT = TL // C


def _softplus(t):
    return jnp.maximum(t, 0.0) + jnp.log1p(jnp.exp(-jnp.abs(t)))


def _fused_kernel(x_ref, qw_ref, kw_ref, vw_ref, gw_ref, gkw1_ref, gkw2_ref,
                  ew_ref, nw_ref, ow_ref, y_ref, s_ref, sb_ref):
    f32 = jnp.float32
    bf16 = jnp.bfloat16

    @pl.when(pl.program_id(0) == 0)
    def _init():
        s_ref[...] = jnp.zeros_like(s_ref)
        sb_ref[...] = jnp.zeros_like(sb_ref)

    x = x_ref[...]
    xb = x.astype(bf16)
    bf = lambda r: r[...].astype(bf16)
    q = jnp.dot(xb, bf(qw_ref), preferred_element_type=f32)
    k = jnp.dot(xb, bf(kw_ref), preferred_element_type=f32)
    v = jnp.dot(xb, bf(vw_ref), preferred_element_type=f32)
    g = jnp.dot(xb, bf(gw_ref), preferred_element_type=f32)
    gkl = jnp.dot(x, gkw1_ref[...], preferred_element_type=f32)
    z = jnp.dot(gkl, gkw2_ref[...], preferred_element_type=f32)
    gk = -_softplus(-z) * (1.0 / GLN)
    el = jnp.dot(x, ew_ref[...], preferred_element_type=f32)
    emax = jnp.max(el, axis=-1, keepdims=True)
    ee = jnp.exp(el - emax)
    e = ee / jnp.sum(ee, axis=-1, keepdims=True)
    w = jnp.max(e, axis=-1, keepdims=True)
    is_max = (e == w).astype(f32)
    tril4 = (jax.lax.broadcasted_iota(jnp.int32, (N, N), 1)
             <= jax.lax.broadcasted_iota(jnp.int32, (N, N), 0))
    prefix = jax.lax.dot_general(is_max, tril4.astype(f32),
                                 (((1,), (1,)), ((), ())),
                                 preferred_element_type=f32)
    m = is_max * (prefix == 1.0).astype(f32)
    kparts = []
    for h in range(H):
        kh = k[:, h * D:(h + 1) * D]
        mx = jnp.max(kh, axis=-1, keepdims=True)
        ex = jnp.exp(kh - mx)
        kparts.append(ex / jnp.sum(ex, axis=-1, keepdims=True))
    qs = q * w
    ks = jnp.concatenate(kparts, axis=1) * w

    row = jax.lax.broadcasted_iota(jnp.int32, (C, C), 0)
    col = jax.lax.broadcasted_iota(jnp.int32, (C, C), 1)
    tril = (col <= row).astype(f32)
    nw = nw_ref[...]

    og_parts = []
    for c in range(CPT):
        r0 = c * C
        m_c = m[r0:r0 + C]
        gk_c = gk[r0:r0 + C]
        same = jax.lax.dot_general(m_c, m_c, (((1,), (1,)), ((), ())),
                                   preferred_element_type=f32)
        M = tril * same
        Gtok = jnp.dot(M, gk_c, preferred_element_type=f32)
        GlastT = jax.lax.dot_general(gk_c, m_c, (((0,), (0,)), ((), ())),
                                     preferred_element_type=f32)
        qd = qs[r0:r0 + C] * jnp.exp(Gtok)
        kd = ks[r0:r0 + C] * jnp.exp(-Gtok)
        eGlT = jnp.exp(GlastT)
        mb = m_c.astype(bf16)
        g_c = g[r0:r0 + C]
        head_outs = []
        for h in range(H):
            sl = slice(h * D, (h + 1) * D)
            qd_h = qd[:, sl]; kd_h = kd[:, sl]
            qb = qd_h.astype(bf16); kb = kd_h.astype(bf16)
            vb = v[r0:r0 + C, sl].astype(bf16)
            A = jax.lax.dot_general(qb, kb, (((1,), (1,)), ((), ())),
                                    preferred_element_type=f32) * M
            o_h = jnp.dot(A.astype(bf16), vb, preferred_element_type=f32)
            o_i = jnp.dot(qb, sb_ref[h], preferred_element_type=f32)
            for n in range(N):
                o_h = o_h + m_c[:, n:n + 1] * o_i[:, n * DV:(n + 1) * DV]
            vn = jnp.concatenate([mb[:, n:n + 1] * vb for n in range(N)],
                                 axis=1)
            dS = jax.lax.dot_general(kb, vn, (((0,), (0,)), ((), ())),
                                     preferred_element_type=f32)
            eg = jnp.repeat(eGlT[sl, :], DV, axis=1)
            s_new = (s_ref[h] + dS) * eg
            s_ref[h] = s_new
            sb_ref[h] = s_new.astype(bf16)
            o_h = o_h * jax.lax.rsqrt(
                jnp.mean(o_h * o_h, axis=-1, keepdims=True) + EPS) * nw
            gh = g_c[:, sl]
            o_h = o_h * (gh * jax.nn.sigmoid(gh))
            head_outs.append(o_h.astype(bf16))
        og_parts.append(jnp.concatenate(head_outs, axis=1))
    og = jnp.concatenate(og_parts, axis=0)
    y_ref[...] = jnp.dot(og, bf(ow_ref), preferred_element_type=f32)


@jax.jit
def kernel(hidden_states, q_w, k_w, v_w, gk_w1, gk_w2, e_w, g_w, norm_w, o_w):
    x = hidden_states[0]
    f32 = jnp.float32
    full = lambda a: pl.BlockSpec(a.shape, lambda i: (0,) * a.ndim)
    rows = lambda c: pl.BlockSpec((TL, c), lambda i: (i, 0))

    y = pl.pallas_call(
        _fused_kernel,
        grid=(L // TL,),
        in_specs=[rows(HID), full(q_w), full(k_w), full(v_w), full(g_w),
                  full(gk_w1), full(gk_w2), full(e_w),
                  full(norm_w.reshape(1, DV)), full(o_w)],
        out_specs=rows(HID),
        out_shape=jax.ShapeDtypeStruct((L, HID), f32),
        scratch_shapes=[pltpu.VMEM((H, D, N * DV), f32),
                        pltpu.VMEM((H, D, N * DV), jnp.bfloat16)],
    )(x, q_w, k_w, v_w, g_w, gk_w1, gk_w2, e_w, norm_w.reshape(1, DV), o_w)
    return y[None]


# speedup vs baseline: 1.1023x; 1.1023x over previous
import jax
import jax.numpy as jnp
from jax.experimental import pallas as pl
from jax.experimental.pallas import tpu as pltpu

L = 2048; HID = 2048; H = 4; D = 128; DV = 128; N = 4; LR = 16
GLN = 16.0; EPS = 1e-5
HD = H * D
C = 512
TL = 512
CPT = TL // C


def _softplus(t):
    return jnp.maximum(t, 0.0) + jnp.log1p(jnp.exp(-jnp.abs(t)))


def _fused_kernel(x_ref, qw_ref, kw_ref, vw_ref, gw_ref, gkew_ref, gkw2_ref,
                  nw_ref, ow_ref, y_ref, s_ref, sb_ref):
    f32 = jnp.float32
    bf16 = jnp.bfloat16

    @pl.when(pl.program_id(0) == 0)
    def _init():
        s_ref[...] = jnp.zeros_like(s_ref)
        sb_ref[...] = jnp.zeros_like(sb_ref)

    x = x_ref[...]
    xb = x.astype(bf16)
    bf = lambda r: r[...].astype(bf16)
    q = jnp.dot(xb, bf(qw_ref), preferred_element_type=f32)
    k = jnp.dot(xb, bf(kw_ref), preferred_element_type=f32)
    v = jnp.dot(xb, bf(vw_ref), preferred_element_type=f32)
    g = jnp.dot(xb, bf(gw_ref), preferred_element_type=f32)
    gkel = jnp.dot(x, gkew_ref[...], preferred_element_type=f32)
    gkl = gkel[:, :LR]
    z = jnp.dot(gkl, gkw2_ref[...], preferred_element_type=f32)
    gk = -_softplus(-z) * (1.0 / GLN)
    el = gkel[:, LR:LR + N]
    emax = jnp.max(el, axis=-1, keepdims=True)
    ee = jnp.exp(el - emax)
    e = ee / jnp.sum(ee, axis=-1, keepdims=True)
    w = jnp.max(e, axis=-1, keepdims=True)
    is_max = (e == w).astype(f32)
    tril4 = (jax.lax.broadcasted_iota(jnp.int32, (N, N), 1)
             <= jax.lax.broadcasted_iota(jnp.int32, (N, N), 0))
    prefix = jax.lax.dot_general(is_max, tril4.astype(f32),
                                 (((1,), (1,)), ((), ())),
                                 preferred_element_type=f32)
    m = is_max * (prefix == 1.0).astype(f32)
    kparts = []
    for h in range(H):
        kh = k[:, h * D:(h + 1) * D]
        mx = jnp.max(kh, axis=-1, keepdims=True)
        ex = jnp.exp(kh - mx)
        kparts.append(ex / jnp.sum(ex, axis=-1, keepdims=True))
    qs = q * w
    ks = jnp.concatenate(kparts, axis=1) * w

    row = jax.lax.broadcasted_iota(jnp.int32, (C, C), 0)
    col = jax.lax.broadcasted_iota(jnp.int32, (C, C), 1)
    tril = (col <= row).astype(f32)
    nw = nw_ref[...]

    og_parts = []
    for c in range(CPT):
        r0 = c * C
        m_c = m[r0:r0 + C]
        gk_c = gk[r0:r0 + C]
        same = jax.lax.dot_general(m_c, m_c, (((1,), (1,)), ((), ())),
                                   preferred_element_type=f32)
        M = tril * same
        Gtok = jnp.dot(M, gk_c, preferred_element_type=f32)
        GlastT = jax.lax.dot_general(gk_c, m_c, (((0,), (0,)), ((), ())),
                                     preferred_element_type=f32)
        qd = qs[r0:r0 + C] * jnp.exp(Gtok)
        kd = ks[r0:r0 + C] * jnp.exp(-Gtok)
        eGlT = jnp.exp(GlastT)
        mb = m_c.astype(bf16)
        g_c = g[r0:r0 + C]
        head_outs = []
        for h in range(H):
            sl = slice(h * D, (h + 1) * D)
            qd_h = qd[:, sl]; kd_h = kd[:, sl]
            qb = qd_h.astype(bf16); kb = kd_h.astype(bf16)
            vb = v[r0:r0 + C, sl].astype(bf16)
            A = jax.lax.dot_general(qb, kb, (((1,), (1,)), ((), ())),
                                    preferred_element_type=f32) * M
            o_h = jnp.dot(A.astype(bf16), vb, preferred_element_type=f32)
            o_i = jnp.dot(qb, sb_ref[h], preferred_element_type=f32)
            for n in range(N):
                o_h = o_h + m_c[:, n:n + 1] * o_i[:, n * DV:(n + 1) * DV]
            vn = jnp.concatenate([mb[:, n:n + 1] * vb for n in range(N)],
                                 axis=1)
            dS = jax.lax.dot_general(kb, vn, (((0,), (0,)), ((), ())),
                                     preferred_element_type=f32)
            eg = jnp.repeat(eGlT[sl, :], DV, axis=1)
            s_new = (s_ref[h] + dS) * eg
            s_ref[h] = s_new
            sb_ref[h] = s_new.astype(bf16)
            o_h = o_h * jax.lax.rsqrt(
                jnp.mean(o_h * o_h, axis=-1, keepdims=True) + EPS) * nw
            gh = g_c[:, sl]
            o_h = o_h * (gh * jax.nn.sigmoid(gh))
            head_outs.append(o_h.astype(bf16))
        og_parts.append(jnp.concatenate(head_outs, axis=1))
    og = jnp.concatenate(og_parts, axis=0)
    y_ref[...] = jnp.dot(og, bf(ow_ref), preferred_element_type=f32)


@jax.jit
def kernel(hidden_states, q_w, k_w, v_w, gk_w1, gk_w2, e_w, g_w, norm_w, o_w):
    x = hidden_states[0]
    f32 = jnp.float32
    full = lambda a: pl.BlockSpec(a.shape, lambda i: (0,) * a.ndim)
    rows = lambda c: pl.BlockSpec((TL, c), lambda i: (i, 0))

    gkew = jnp.concatenate([gk_w1, e_w], axis=1)
    y = pl.pallas_call(
        _fused_kernel,
        grid=(L // TL,),
        in_specs=[rows(HID), full(q_w), full(k_w), full(v_w), full(g_w),
                  full(gkew), full(gk_w2),
                  full(norm_w.reshape(1, DV)), full(o_w)],
        out_specs=rows(HID),
        out_shape=jax.ShapeDtypeStruct((L, HID), f32),
        scratch_shapes=[pltpu.VMEM((H, D, N * DV), f32),
                        pltpu.VMEM((H, D, N * DV), jnp.bfloat16)],
    )(x, q_w, k_w, v_w, g_w, gkew, gk_w2, norm_w.reshape(1, DV), o_w)
    return y[None]


# speedup vs baseline: 1.1041x; 1.0016x over previous
import jax
import jax.numpy as jnp
from jax.experimental import pallas as pl
from jax.experimental.pallas import tpu as pltpu

L = 2048; HID = 2048; H = 4; D = 128; DV = 128; N = 4; LR = 16
GLN = 16.0; EPS = 1e-5
HD = H * D
C = 512
TL = 512
CPT = TL // C


def _softplus(t):
    return jnp.maximum(t, 0.0) + jnp.log1p(jnp.exp(-jnp.abs(t)))


def _fused_kernel(x_ref, qw_ref, kw_ref, vw_ref, gw_ref, gkew_ref, gkw2_ref,
                  nw_ref, ow_ref, y_ref, s_ref, sb_ref, tril_ref):
    f32 = jnp.float32
    bf16 = jnp.bfloat16

    @pl.when(pl.program_id(0) == 0)
    def _init():
        s_ref[...] = jnp.zeros_like(s_ref)
        sb_ref[...] = jnp.zeros_like(sb_ref)
        row = jax.lax.broadcasted_iota(jnp.int32, (C, C), 0)
        col = jax.lax.broadcasted_iota(jnp.int32, (C, C), 1)
        tril_ref[...] = (col <= row).astype(f32)

    x = x_ref[...]
    xb = x.astype(bf16)
    bf = lambda r: r[...].astype(bf16)
    q = jnp.dot(xb, bf(qw_ref), preferred_element_type=f32)
    k = jnp.dot(xb, bf(kw_ref), preferred_element_type=f32)
    v = jnp.dot(xb, bf(vw_ref), preferred_element_type=f32)
    g = jnp.dot(xb, bf(gw_ref), preferred_element_type=f32)
    gkel = jnp.dot(x, gkew_ref[...], preferred_element_type=f32)
    gkl = gkel[:, :LR]
    z = jnp.dot(gkl, gkw2_ref[...], preferred_element_type=f32)
    gk = -_softplus(-z) * (1.0 / GLN)
    el = gkel[:, LR:LR + N]
    emax = jnp.max(el, axis=-1, keepdims=True)
    ee = jnp.exp(el - emax)
    e = ee / jnp.sum(ee, axis=-1, keepdims=True)
    w = jnp.max(e, axis=-1, keepdims=True)
    is_max = (e == w).astype(f32)
    tril4 = (jax.lax.broadcasted_iota(jnp.int32, (N, N), 1)
             <= jax.lax.broadcasted_iota(jnp.int32, (N, N), 0))
    prefix = jax.lax.dot_general(is_max, tril4.astype(f32),
                                 (((1,), (1,)), ((), ())),
                                 preferred_element_type=f32)
    m = is_max * (prefix == 1.0).astype(f32)
    kparts = []
    for h in range(H):
        kh = k[:, h * D:(h + 1) * D]
        mx = jnp.max(kh, axis=-1, keepdims=True)
        ex = jnp.exp(kh - mx)
        kparts.append(ex / jnp.sum(ex, axis=-1, keepdims=True))
    qs = q * w
    ks = jnp.concatenate(kparts, axis=1) * w

    tril = tril_ref[...]
    nw = nw_ref[...]

    og_parts = []
    for c in range(CPT):
        r0 = c * C
        m_c = m[r0:r0 + C]
        gk_c = gk[r0:r0 + C]
        same = jax.lax.dot_general(m_c, m_c, (((1,), (1,)), ((), ())),
                                   preferred_element_type=f32)
        M = tril * same
        Gtok = jnp.dot(M, gk_c, preferred_element_type=f32)
        GlastT = jax.lax.dot_general(gk_c, m_c, (((0,), (0,)), ((), ())),
                                     preferred_element_type=f32)
        qd = qs[r0:r0 + C] * jnp.exp(Gtok)
        kd = ks[r0:r0 + C] * jnp.exp(-Gtok)
        eGlT = jnp.exp(GlastT)
        mb = m_c.astype(bf16)
        g_c = g[r0:r0 + C]
        head_outs = []
        for h in range(H):
            sl = slice(h * D, (h + 1) * D)
            qd_h = qd[:, sl]; kd_h = kd[:, sl]
            qb = qd_h.astype(bf16); kb = kd_h.astype(bf16)
            vb = v[r0:r0 + C, sl].astype(bf16)
            A = jax.lax.dot_general(qb, kb, (((1,), (1,)), ((), ())),
                                    preferred_element_type=f32) * M
            o_h = jnp.dot(A.astype(bf16), vb, preferred_element_type=f32)
            o_i = jnp.dot(qb, sb_ref[h], preferred_element_type=f32)
            for n in range(N):
                o_h = o_h + m_c[:, n:n + 1] * o_i[:, n * DV:(n + 1) * DV]
            vn = jnp.concatenate([mb[:, n:n + 1] * vb for n in range(N)],
                                 axis=1)
            dS = jax.lax.dot_general(kb, vn, (((0,), (0,)), ((), ())),
                                     preferred_element_type=f32)
            eg = jnp.repeat(eGlT[sl, :], DV, axis=1)
            s_new = (s_ref[h] + dS) * eg
            s_ref[h] = s_new
            sb_ref[h] = s_new.astype(bf16)
            o_h = o_h * jax.lax.rsqrt(
                jnp.mean(o_h * o_h, axis=-1, keepdims=True) + EPS) * nw
            gh = g_c[:, sl]
            o_h = o_h * (gh * jax.nn.sigmoid(gh))
            head_outs.append(o_h.astype(bf16))
        og_parts.append(jnp.concatenate(head_outs, axis=1))
    og = jnp.concatenate(og_parts, axis=0)
    y_ref[...] = jnp.dot(og, bf(ow_ref), preferred_element_type=f32)


@jax.jit
def kernel(hidden_states, q_w, k_w, v_w, gk_w1, gk_w2, e_w, g_w, norm_w, o_w):
    x = hidden_states[0]
    f32 = jnp.float32
    full = lambda a: pl.BlockSpec(a.shape, lambda i: (0,) * a.ndim)
    rows = lambda c: pl.BlockSpec((TL, c), lambda i: (i, 0))

    gkew = jnp.concatenate([gk_w1, e_w], axis=1)
    y = pl.pallas_call(
        _fused_kernel,
        grid=(L // TL,),
        in_specs=[rows(HID), full(q_w), full(k_w), full(v_w), full(g_w),
                  full(gkew), full(gk_w2),
                  full(norm_w.reshape(1, DV)), full(o_w)],
        out_specs=rows(HID),
        out_shape=jax.ShapeDtypeStruct((L, HID), f32),
        scratch_shapes=[pltpu.VMEM((H, D, N * DV), f32),
                        pltpu.VMEM((H, D, N * DV), jnp.bfloat16),
                        pltpu.VMEM((C, C), f32)],
    )(x, q_w, k_w, v_w, g_w, gkew, gk_w2, norm_w.reshape(1, DV), o_w)
    return y[None]
